# baseline (speedup 1.0000x reference)
import functools
import math

import jax
import jax.numpy as jnp
from jax import lax
from jax.experimental import pallas as pl
from jax.experimental.pallas import tpu as pltpu
from jax.experimental.pallas import tpu_sc as plsc

_NC = 2
_NS = 16
_NW = _NC * _NS


@functools.cache
def _build(B, V, D, C, K):
  assert C == K * 128 and B % (_NW * C) == 0 and D % 16 == 0
  bpw = B // _NW
  G = bpw // C
  scale = math.sqrt(D)

  mesh = plsc.VectorSubcoreMesh(core_axis_name="c", subcore_axis_name="s")

  @functools.partial(
      pl.kernel,
      out_type=jax.ShapeDtypeStruct((B, D), jnp.float32),
      mesh=mesh,
      scratch_types=[
          pltpu.VMEM((K, 128), jnp.int32),
          pltpu.VMEM((C, D), jnp.float32),
          pltpu.SemaphoreType.DMA,
      ],
      compiler_params=pltpu.CompilerParams(use_tc_tiling_on_sc=False),
  )
  def emb_kernel(idx_hbm, table_hbm, out_hbm, idx_v, rows_v, sem):
    wid = lax.axis_index("s") * _NC + lax.axis_index("c")
    row0 = wid * bpw

    def chunk(g):
      off = row0 + g * C
      irow = pl.multiple_of(off // 128, 8)
      pltpu.sync_copy(idx_hbm.at[pl.ds(irow, K)], idx_v)
      copies = [
          pltpu.async_copy(
              table_hbm.at[idx_v.at[j]],
              rows_v.at[pl.ds(j * 128, 128)],
              sem,
          )
          for j in range(K)
      ]
      for c in copies:
        c.wait()

      @plsc.parallel_loop(0, C, 1, unroll=4)
      def _scale(r):
        for j in range(D // 16):
          sl = pl.ds(j * 16, 16)
          rows_v[r, sl] = rows_v[r, sl] * scale

      pltpu.sync_copy(rows_v, out_hbm.at[pl.ds(off, C)])

    pl.loop(0, G)(chunk)

  return emb_kernel


def kernel(x, table):
  Bb, S = x.shape
  V, D = table.shape
  B = Bb * S
  xf = x.reshape(B // 128, 128).astype(jnp.int32)
  out = _build(B, V, D, 1024, 8)(xf, table)
  return out.reshape(Bb, S, D)


# speedup vs baseline: 1.0071x; 1.0071x over previous
import functools
import math

import jax
import jax.numpy as jnp
from jax import lax
from jax.experimental import pallas as pl
from jax.experimental.pallas import tpu as pltpu
from jax.experimental.pallas import tpu_sc as plsc

_NC = 2
_NS = 16
_NW = _NC * _NS


@functools.cache
def _build(B, V, D, C):
  assert B % (_NW * C) == 0 and D % 16 == 0 and C % 8 == 0
  bpw = B // _NW
  G = bpw // C
  scale = math.sqrt(D)

  mesh = plsc.VectorSubcoreMesh(core_axis_name="c", subcore_axis_name="s")

  @functools.partial(
      pl.kernel,
      out_type=jax.ShapeDtypeStruct((B, D), jnp.float32),
      mesh=mesh,
      scratch_types=[
          pltpu.VMEM((C,), jnp.int32),
          pltpu.VMEM((C, D), jnp.float32),
          pltpu.SemaphoreType.DMA,
      ],
      compiler_params=pltpu.CompilerParams(use_tc_tiling_on_sc=False),
  )
  def emb_kernel(idx_hbm, table_hbm, out_hbm, idx_v, rows_v, sem):
    wid = lax.axis_index("s") * _NC + lax.axis_index("c")
    row0 = wid * bpw

    def chunk(g):
      off = row0 + g * C
      pltpu.sync_copy(idx_hbm.at[pl.ds(off, C)], idx_v)
      pltpu.async_copy(table_hbm.at[idx_v], rows_v, sem).wait()

      @plsc.parallel_loop(0, C, 1, unroll=4)
      def _scale(r):
        for j in range(D // 16):
          sl = pl.ds(j * 16, 16)
          rows_v[r, sl] = rows_v[r, sl] * scale

      pltpu.sync_copy(rows_v, out_hbm.at[pl.ds(off, C)])

    pl.loop(0, G)(chunk)

  return emb_kernel


def kernel(x, table):
  Bb, S = x.shape
  V, D = table.shape
  B = Bb * S
  xf = x.reshape(B).astype(jnp.int32)
  out = _build(B, V, D, 1600)(xf, table)
  return out.reshape(Bb, S, D)
